# baseline (speedup 1.0000x reference)
import functools

import jax
import jax.numpy as jnp
from jax import lax
from jax.experimental import pallas as pl
from jax.experimental.pallas import tpu as pltpu
from jax.experimental.pallas import tpu_sc as plsc

_NC = 2
_NS = 16
_NW = _NC * _NS
_LANES = 128
_K = 2
_HALVES = 2


def _relmm_body(h_ref, w_ref, t_ref):
    t_ref[...] = jnp.dot(h_ref[...], w_ref[0],
                         preferred_element_type=jnp.float32)


def _rel_transform(h, W, bn=10000):
    n, d = h.shape
    r = W.shape[0]
    nb = n // bn
    return pl.pallas_call(
        _relmm_body,
        grid=(nb, r),
        in_specs=[
            pl.BlockSpec((bn, d), lambda i, j: (i, 0)),
            pl.BlockSpec((1, d, d), lambda i, j: (j, 0, 0)),
        ],
        out_specs=pl.BlockSpec((bn, d), lambda i, j: (j * nb + i, 0)),
        out_shape=jax.ShapeDtypeStruct((r * n, d), jnp.float32),
    )(h, W)


def _combine_body(p0_ref, p1_ref, h_ref, wl_ref, b_ref, o_ref, *, act):
    out = (p0_ref[...] + p1_ref[...]
           + jnp.dot(h_ref[...], wl_ref[...],
                     preferred_element_type=jnp.float32)
           + b_ref[...])
    if act:
        out = jnp.maximum(out, 0.0)
    o_ref[...] = out


def _combine(p0, p1, h, Wl, b, act, bn=5000):
    n, d = h.shape
    nb = n // bn
    return pl.pallas_call(
        functools.partial(_combine_body, act=act),
        grid=(nb,),
        in_specs=[
            pl.BlockSpec((bn, d), lambda i: (i, 0)),
            pl.BlockSpec((bn, d), lambda i: (i, 0)),
            pl.BlockSpec((bn, d), lambda i: (i, 0)),
            pl.BlockSpec((d, d), lambda i: (0, 0)),
            pl.BlockSpec((1, d), lambda i: (0, 0)),
        ],
        out_specs=pl.BlockSpec((bn, d), lambda i: (i, 0)),
        out_shape=jax.ShapeDtypeStruct((n, d), jnp.float32),
    )(p0, p1, h, Wl, b.reshape(1, d))


def _sc_edge_pass(t, gidx2, dst2, zinit, npad):
    erows = gidx2.shape[0]
    d = t.shape[1]
    rows_per_w = erows // _NW
    half = rows_per_w // _HALVES
    rows_per_tile = npad // _NS

    mesh = plsc.VectorSubcoreMesh(core_axis_name="c", subcore_axis_name="s",
                                  num_cores=_NC, num_subcores=_NS)

    @functools.partial(
        pl.kernel,
        out_type=(jax.ShapeDtypeStruct((npad, d), jnp.float32),
                  jax.ShapeDtypeStruct((npad, d), jnp.float32)),
        mesh=mesh,
        scratch_types=[
            pltpu.VMEM((half, _LANES), jnp.int32),
            pltpu.VMEM((half, _LANES), jnp.int32),
            pltpu.VMEM((_K * _LANES, d), jnp.float32),
            pltpu.VMEM_SHARED((npad, d), jnp.float32),
            pltpu.SemaphoreType.DMA,
            pltpu.SemaphoreType.DMA,
        ],
    )
    def k(t_hbm, g_hbm, d_hbm, z_hbm, out0, out1, gbuf, dbuf, ring, acc,
          gsem, ssem):
        c = lax.axis_index("c")
        s = lax.axis_index("s")
        wid = s * _NC + c
        zb = s * rows_per_tile
        base = wid * rows_per_w

        def slot_of(j):
            return (j % _K) * _LANES

        def gather_wait(j):
            pltpu.make_async_copy(
                t_hbm.at[gbuf.at[j]],
                ring.at[pl.ds(slot_of(j), _LANES)], gsem).wait()

        def scatter_fire(j):
            pltpu.async_copy(ring.at[pl.ds(slot_of(j), _LANES)],
                             acc.at[dbuf.at[j]], ssem, add=True)

        def scatter_wait(j):
            pltpu.make_async_copy(ring.at[pl.ds(slot_of(j), _LANES)],
                                  acc.at[dbuf.at[j]], ssem).wait()

        for hh in range(_HALVES):
            h0 = base + hh * half
            pltpu.sync_copy(g_hbm.at[pl.ds(h0, half)], gbuf)
            pltpu.sync_copy(d_hbm.at[pl.ds(h0, half)], dbuf)
            for j in range(_K):
                pltpu.async_copy(t_hbm.at[gbuf.at[j]],
                                 ring.at[pl.ds(slot_of(j), _LANES)], gsem)
            if hh == 0:
                pltpu.sync_copy(z_hbm, acc.at[pl.ds(zb, rows_per_tile)])
                plsc.subcore_barrier()
            gather_wait(0)
            scatter_fire(0)

            def step(jm1, carry):
                j = jm1 + 1
                scatter_wait(j - 1)

                @pl.when(j + 1 < half)
                def _():
                    pltpu.async_copy(
                        t_hbm.at[gbuf.at[j + 1]],
                        ring.at[pl.ds(slot_of(j + 1), _LANES)], gsem)

                gather_wait(j)
                scatter_fire(j)
                return carry

            lax.fori_loop(0, half - 1, step, 0)
            scatter_wait(half - 1)
        plsc.subcore_barrier()

        @pl.when(c == 0)
        def _():
            pltpu.sync_copy(acc.at[pl.ds(zb, rows_per_tile)],
                            out0.at[pl.ds(zb, rows_per_tile)])

        @pl.when(c == 1)
        def _():
            pltpu.sync_copy(acc.at[pl.ds(zb, rows_per_tile)],
                            out1.at[pl.ds(zb, rows_per_tile)])

    return k(t, gidx2, dst2, zinit)


def kernel(x, edge_index, edge_type, W0, Wl0, b0, W1, Wl1, b1, W2, Wl2, b2):
    n, d = x.shape
    e = edge_type.shape[0]

    src = edge_index[0]
    dst = edge_index[1]
    gidx = edge_type * n + src

    npad = ((n + _NS) + (_NS * 8) - 1) // (_NS * 8) * (_NS * 8)
    zinit = jnp.zeros((npad // _NS, d), jnp.float32)

    group = _LANES * _HALVES * _K * _NW
    ep = ((e + group - 1) // group) * group
    erows = ep // _LANES
    npd = ep - e
    pad_cycle = jnp.arange(npd, dtype=jnp.int32)
    gidx2 = jnp.concatenate(
        [gidx, pad_cycle % _LANES]).reshape(erows, _LANES)
    dst2 = jnp.concatenate(
        [dst, n + pad_cycle % (npad - n)]).reshape(erows, _LANES)

    h = x
    for (W, Wl, b, act) in ((W0, Wl0, b0, True),
                            (W1, Wl1, b1, True),
                            (W2, Wl2, b2, False)):
        t = _rel_transform(h, W)
        p0, p1 = _sc_edge_pass(t, gidx2, dst2, zinit, npad)
        h = _combine(p0, p1, h, Wl, b, act)
    return h


# speedup vs baseline: 1.0135x; 1.0135x over previous
import functools

import jax
import jax.numpy as jnp
from jax import lax
from jax.experimental import pallas as pl
from jax.experimental.pallas import tpu as pltpu
from jax.experimental.pallas import tpu_sc as plsc

_NC = 2
_NS = 16
_NW = _NC * _NS
_LANES = 128
_K = 2
_HALVES = 2


def _relmm_body(h_ref, w_ref, t_ref):
    t_ref[...] = jnp.dot(h_ref[...], w_ref[0],
                         preferred_element_type=jnp.float32)


def _rel_transform(h, W, bn=10000):
    n, d = h.shape
    r = W.shape[0]
    nb = n // bn
    return pl.pallas_call(
        _relmm_body,
        grid=(nb, r),
        in_specs=[
            pl.BlockSpec((bn, d), lambda i, j: (i, 0)),
            pl.BlockSpec((1, d, d), lambda i, j: (j, 0, 0)),
        ],
        out_specs=pl.BlockSpec((bn, d), lambda i, j: (j * nb + i, 0)),
        out_shape=jax.ShapeDtypeStruct((r * n, d), jnp.float32),
    )(h, W)


def _combine_body(p0_ref, p1_ref, h_ref, wl_ref, b_ref, o_ref, *, act):
    out = (p0_ref[...] + p1_ref[...]
           + jnp.dot(h_ref[...], wl_ref[...],
                     preferred_element_type=jnp.float32)
           + b_ref[...])
    if act:
        out = jnp.maximum(out, 0.0)
    o_ref[...] = out


def _combine(p0, p1, h, Wl, b, act, bn=5000):
    n, d = h.shape
    nb = n // bn
    return pl.pallas_call(
        functools.partial(_combine_body, act=act),
        grid=(nb,),
        in_specs=[
            pl.BlockSpec((bn, d), lambda i: (i, 0)),
            pl.BlockSpec((bn, d), lambda i: (i, 0)),
            pl.BlockSpec((bn, d), lambda i: (i, 0)),
            pl.BlockSpec((d, d), lambda i: (0, 0)),
            pl.BlockSpec((1, d), lambda i: (0, 0)),
        ],
        out_specs=pl.BlockSpec((bn, d), lambda i: (i, 0)),
        out_shape=jax.ShapeDtypeStruct((n, d), jnp.float32),
    )(p0, p1, h, Wl, b.reshape(1, d))


def _fused_body(p0_ref, p1_ref, h_ref, wl_ref, b_ref, wn_ref,
                t_ref, ho_ref, out_scr):
    r = pl.program_id(0)

    @pl.when(r == 0)
    def _():
        out = jnp.maximum(
            p0_ref[...] + p1_ref[...]
            + jnp.dot(h_ref[...], wl_ref[...],
                      preferred_element_type=jnp.float32)
            + b_ref[...], 0.0)
        out_scr[...] = out
        ho_ref[...] = out

    t_ref[...] = jnp.dot(out_scr[...], wn_ref[0],
                         preferred_element_type=jnp.float32)


def _combine_transform(p0, p1, h, Wl, b, Wn):
    n, d = h.shape
    r = Wn.shape[0]
    return pl.pallas_call(
        _fused_body,
        grid=(r,),
        in_specs=[
            pl.BlockSpec((n, d), lambda j: (0, 0)),
            pl.BlockSpec((n, d), lambda j: (0, 0)),
            pl.BlockSpec((n, d), lambda j: (0, 0)),
            pl.BlockSpec((d, d), lambda j: (0, 0)),
            pl.BlockSpec((1, d), lambda j: (0, 0)),
            pl.BlockSpec((1, d, d), lambda j: (j, 0, 0)),
        ],
        out_specs=(pl.BlockSpec((n, d), lambda j: (j, 0)),
                   pl.BlockSpec((n, d), lambda j: (0, 0))),
        out_shape=(jax.ShapeDtypeStruct((r * n, d), jnp.float32),
                   jax.ShapeDtypeStruct((n, d), jnp.float32)),
        scratch_shapes=[pltpu.VMEM((n, d), jnp.float32)],
    )(p0, p1, h, Wl, b.reshape(1, d), Wn)


def _sc_edge_pass(t, gidx2, dst2, zinit, npad):
    erows = gidx2.shape[0]
    d = t.shape[1]
    rows_per_w = erows // _NW
    half = rows_per_w // _HALVES
    rows_per_tile = npad // _NS

    mesh = plsc.VectorSubcoreMesh(core_axis_name="c", subcore_axis_name="s",
                                  num_cores=_NC, num_subcores=_NS)

    @functools.partial(
        pl.kernel,
        out_type=(jax.ShapeDtypeStruct((npad, d), jnp.float32),
                  jax.ShapeDtypeStruct((npad, d), jnp.float32)),
        mesh=mesh,
        scratch_types=[
            pltpu.VMEM((half, _LANES), jnp.int32),
            pltpu.VMEM((half, _LANES), jnp.int32),
            pltpu.VMEM((_K * _LANES, d), jnp.float32),
            pltpu.VMEM_SHARED((npad, d), jnp.float32),
            pltpu.SemaphoreType.DMA,
            pltpu.SemaphoreType.DMA,
        ],
    )
    def k(t_hbm, g_hbm, d_hbm, z_hbm, out0, out1, gbuf, dbuf, ring, acc,
          gsem, ssem):
        c = lax.axis_index("c")
        s = lax.axis_index("s")
        wid = s * _NC + c
        zb = s * rows_per_tile
        base = wid * rows_per_w

        def slot_of(j):
            return (j % _K) * _LANES

        def gather_wait(j):
            pltpu.make_async_copy(
                t_hbm.at[gbuf.at[j]],
                ring.at[pl.ds(slot_of(j), _LANES)], gsem).wait()

        def scatter_fire(j):
            pltpu.async_copy(ring.at[pl.ds(slot_of(j), _LANES)],
                             acc.at[dbuf.at[j]], ssem, add=True)

        def scatter_wait(j):
            pltpu.make_async_copy(ring.at[pl.ds(slot_of(j), _LANES)],
                                  acc.at[dbuf.at[j]], ssem).wait()

        for hh in range(_HALVES):
            h0 = base + hh * half
            pltpu.sync_copy(g_hbm.at[pl.ds(h0, half)], gbuf)
            pltpu.sync_copy(d_hbm.at[pl.ds(h0, half)], dbuf)
            for j in range(_K):
                pltpu.async_copy(t_hbm.at[gbuf.at[j]],
                                 ring.at[pl.ds(slot_of(j), _LANES)], gsem)
            if hh == 0:
                pltpu.sync_copy(z_hbm, acc.at[pl.ds(zb, rows_per_tile)])
                plsc.subcore_barrier()
            gather_wait(0)
            scatter_fire(0)

            def step(jm1, carry):
                j = jm1 + 1
                scatter_wait(j - 1)

                @pl.when(j + 1 < half)
                def _():
                    pltpu.async_copy(
                        t_hbm.at[gbuf.at[j + 1]],
                        ring.at[pl.ds(slot_of(j + 1), _LANES)], gsem)

                gather_wait(j)
                scatter_fire(j)
                return carry

            lax.fori_loop(0, half - 1, step, 0)
            scatter_wait(half - 1)
        plsc.subcore_barrier()

        @pl.when(c == 0)
        def _():
            pltpu.sync_copy(acc.at[pl.ds(zb, rows_per_tile)],
                            out0.at[pl.ds(zb, rows_per_tile)])

        @pl.when(c == 1)
        def _():
            pltpu.sync_copy(acc.at[pl.ds(zb, rows_per_tile)],
                            out1.at[pl.ds(zb, rows_per_tile)])

    return k(t, gidx2, dst2, zinit)


def kernel(x, edge_index, edge_type, W0, Wl0, b0, W1, Wl1, b1, W2, Wl2, b2):
    n, d = x.shape
    e = edge_type.shape[0]

    src = edge_index[0]
    dst = edge_index[1]
    gidx = edge_type * n + src

    npad = ((n + _NS) + (_NS * 8) - 1) // (_NS * 8) * (_NS * 8)
    zinit = jnp.zeros((npad // _NS, d), jnp.float32)

    group = _LANES * _HALVES * _K * _NW
    ep = ((e + group - 1) // group) * group
    erows = ep // _LANES
    npd = ep - e
    pad_cycle = jnp.arange(npd, dtype=jnp.int32)
    gidx2 = jnp.concatenate(
        [gidx, pad_cycle % _LANES]).reshape(erows, _LANES)
    dst2 = jnp.concatenate(
        [dst, n + pad_cycle % (npad - n)]).reshape(erows, _LANES)

    t = _rel_transform(x, W0)
    p0, p1 = _sc_edge_pass(t, gidx2, dst2, zinit, npad)
    t, h = _combine_transform(p0, p1, x, Wl0, b0, W1)
    p0, p1 = _sc_edge_pass(t, gidx2, dst2, zinit, npad)
    t, h = _combine_transform(p0, p1, h, Wl1, b1, W2)
    p0, p1 = _sc_edge_pass(t, gidx2, dst2, zinit, npad)
    return _combine(p0, p1, h, Wl2, b2, False)
